# baseline (speedup 1.0000x reference)
import functools

import jax
import jax.numpy as jnp
from jax import lax
from jax.experimental import pallas as pl
from jax.experimental.pallas import tpu as pltpu
from jax.experimental.pallas import tpu_sc as plsc

NC, NS = 2, 16
NW = NC * NS
N = 10000
NP = 10240
E = 320000
CH = 128
NCHUNK = 80
NSUB = 8
TOTC = NW * NCHUNK
N0 = 160
N1 = 160 - N0
EPT = NCHUNK * CH
EPAD = NW * EPT - E
D = 128
DW = 16
DUMMY_DST = 10008
RPS = NP // NS
BR = 512


def _make_agg(width):
  mesh = plsc.VectorSubcoreMesh(
      core_axis_name="c", subcore_axis_name="s", num_cores=NC,
      num_subcores=NS)

  out_type = jax.ShapeDtypeStruct((NC, NP, width), jnp.float32)
  scratch = [
      pltpu.VMEM((NSUB, CH), jnp.int32),
      pltpu.VMEM((NSUB, CH), jnp.int32),
      pltpu.VMEM((CH, width), jnp.float32),
      pltpu.VMEM((CH, width), jnp.float32),
      pltpu.VMEM_SHARED((NP, width), jnp.float32),
      pltpu.SemaphoreType.DMA,
      pltpu.SemaphoreType.DMA,
      pltpu.SemaphoreType.DMA,
      pltpu.SemaphoreType.DMA,
  ]

  def body(x_hbm, src_hbm, dst_hbm, agg_hbm, src_v, dst_v, rows0, rows1,
           agg_sh, gs0, gs1, ss0, ss1):
    c = lax.axis_index("c")
    s = lax.axis_index("s")
    wid = s * NC + c
    rows = (rows0, rows1)
    gs = (gs0, gs1)
    ss = (ss0, ss1)

    zeros16 = jnp.zeros((16,), jnp.float32)

    def zrow(i, carry):
      for k in range(width // 16):
        rows0[i, pl.ds(k * 16, 16)] = zeros16
      return carry

    lax.fori_loop(0, CH, zrow, 0)

    def zagg(i, carry):
      pltpu.sync_copy(rows0, agg_sh.at[pl.ds(s * RPS + i * CH, CH)])
      return carry

    lax.fori_loop(0, RPS // CH, zagg, 0)
    plsc.subcore_barrier()

    def mk_sup(base):
      def sup(i, carry):
        off = base + i * NSUB
        pltpu.sync_copy(src_hbm.at[pl.ds(off, NSUB)], src_v)
        pltpu.sync_copy(dst_hbm.at[pl.ds(off, NSUB)], dst_v)
        gd = [None] * NSUB
        gd[0] = pltpu.async_copy(x_hbm.at[src_v.at[0]], rows[0], gs[0])
        gd[1] = pltpu.async_copy(x_hbm.at[src_v.at[1]], rows[1], gs[1])
        for k in range(NSUB):
          b = k % 2
          gd[k].wait()
          sd = pltpu.async_copy(rows[b], agg_sh.at[dst_v.at[k]], ss[b],
                                add=True)
          sd.wait()
          if k + 2 < NSUB:
            gd[k + 2] = pltpu.async_copy(x_hbm.at[src_v.at[k + 2]], rows[b],
                                         gs[b])
        return carry
      return sup

    @pl.when(c == 0)
    def _():
      lax.fori_loop(0, N0 // NSUB, mk_sup(s * N0), 0)

    @pl.when(c == 1)
    def _():
      lax.fori_loop(0, N1 // NSUB, mk_sup(NS * N0 + s * N1), 0)

    plsc.subcore_barrier()

    pltpu.sync_copy(agg_sh.at[pl.ds(s * RPS, RPS)],
                    agg_hbm.at[c, pl.ds(s * RPS, RPS)])

  return pl.kernel(
      body, out_type=out_type, mesh=mesh, scratch_types=scratch)


_agg = _make_agg(D)


def _make_deg():
  mesh = plsc.VectorSubcoreMesh(
      core_axis_name="c", subcore_axis_name="s", num_cores=NC,
      num_subcores=NS)

  out_type = jax.ShapeDtypeStruct((NC, NP, D), jnp.float32)
  scratch = [
      pltpu.VMEM((NSUB, CH), jnp.int32),
      pltpu.VMEM((CH, D), jnp.float32),
      pltpu.VMEM_SHARED((NP, D), jnp.float32),
      pltpu.SemaphoreType.DMA,
  ]

  def body(dst_hbm, deg_hbm, dst_v, ones_v, deg_sh, sem):
    c = lax.axis_index("c")
    s = lax.axis_index("s")
    wid = s * NC + c

    zeros16 = jnp.zeros((16,), jnp.float32)
    ones16 = jnp.ones((16,), jnp.float32)

    def zrow(i, carry):
      for k in range(D // 16):
        ones_v[i, pl.ds(k * 16, 16)] = zeros16
      return carry

    lax.fori_loop(0, CH, zrow, 0)

    def zdeg(i, carry):
      pltpu.sync_copy(ones_v, deg_sh.at[pl.ds(s * RPS + i * CH, CH)])
      return carry

    lax.fori_loop(0, RPS // CH, zdeg, 0)

    def frow(i, carry):
      for k in range(D // 16):
        ones_v[i, pl.ds(k * 16, 16)] = ones16
      return carry

    lax.fori_loop(0, CH, frow, 0)
    plsc.subcore_barrier()

    def sup(i, carry):
      pltpu.sync_copy(dst_hbm.at[pl.ds(wid * NCHUNK + i * NSUB, NSUB)], dst_v)
      descs = [
          pltpu.async_copy(ones_v, deg_sh.at[dst_v.at[k]], sem, add=True)
          for k in range(NSUB)
      ]
      for d in descs:
        d.wait()
      return carry

    lax.fori_loop(0, NCHUNK // NSUB, sup, 0)
    plsc.subcore_barrier()

    pltpu.sync_copy(deg_sh.at[pl.ds(s * RPS, RPS)],
                    deg_hbm.at[c, pl.ds(s * RPS, RPS)])

  return pl.kernel(
      body, out_type=out_type, mesh=mesh, scratch_types=scratch)


_deg = _make_deg()


def _layer1_body(aggp_ref, degp_ref, x_ref, wl_ref, bl_ref, wr_ref, out_ref):
  agg = aggp_ref[0] + aggp_ref[1]
  deg = degp_ref[0, :, 0] + degp_ref[1, :, 0]
  inv = 1.0 / jnp.clip(deg, 1.0, None)
  mean = agg * inv[:, None]
  h = (jnp.dot(mean, wl_ref[...], preferred_element_type=jnp.float32)
       + bl_ref[...]
       + jnp.dot(x_ref[...], wr_ref[...], preferred_element_type=jnp.float32))
  out_ref[...] = jnp.maximum(h, 0.0)


def _layer2_body(aggp_ref, degp_ref, x_ref, wl_ref, bl_ref, wr_ref,
                 wo_ref, bo_ref, out_ref):
  agg = aggp_ref[0] + aggp_ref[1]
  deg = degp_ref[0, :, 0] + degp_ref[1, :, 0]
  inv = 1.0 / jnp.clip(deg, 1.0, None)
  mean = agg * inv[:, None]
  h = (jnp.dot(mean, wl_ref[...], preferred_element_type=jnp.float32)
       + bl_ref[...]
       + jnp.dot(x_ref[...], wr_ref[...], preferred_element_type=jnp.float32))
  h = jnp.maximum(h, 0.0)
  out_ref[...] = (jnp.dot(h, wo_ref[...], preferred_element_type=jnp.float32)
                  + bo_ref[...])


def _tc_layer(body, n_extra):
  grid = (NP // BR,)
  in_specs = [
      pl.BlockSpec((NC, BR, D), lambda i: (0, i, 0)),
      pl.BlockSpec((NC, BR, D), lambda i: (0, i, 0)),
      pl.BlockSpec((BR, D), lambda i: (i, 0)),
      pl.BlockSpec((D, D), lambda i: (0, 0)),
      pl.BlockSpec((1, D), lambda i: (0, 0)),
      pl.BlockSpec((D, D), lambda i: (0, 0)),
  ] + [
      pl.BlockSpec((D, D), lambda i: (0, 0)),
      pl.BlockSpec((1, D), lambda i: (0, 0)),
  ][:n_extra]
  return pl.pallas_call(
      body,
      grid=grid,
      in_specs=in_specs,
      out_specs=pl.BlockSpec((BR, D), lambda i: (i, 0)),
      out_shape=jax.ShapeDtypeStruct((NP, D), jnp.float32),
  )


_tc1 = _tc_layer(_layer1_body, 0)
_tc2 = _tc_layer(_layer2_body, 2)


def kernel(x_content, edge_index, edge_type, W_l1, b_l1, W_r1,
           W_l2, b_l2, W_r2, W_out, b_out):
  ei = edge_index.astype(jnp.int32)
  src = jnp.concatenate(
      [ei[0], jnp.zeros((EPAD,), jnp.int32)]).reshape(TOTC, CH)
  dst = jnp.concatenate(
      [ei[1], jnp.full((EPAD,), DUMMY_DST, jnp.int32)]).reshape(TOTC, CH)
  xp = jnp.pad(x_content, ((0, NP - N), (0, 0)))

  degp = _deg(dst)
  aggp1 = _agg(xp, src, dst)
  h1 = _tc1(aggp1, degp, xp, W_l1.T, b_l1[None, :], W_r1.T)
  aggp2 = _agg(h1, src, dst)

  wo = jnp.zeros((D, D), jnp.float32).at[:, :W_out.shape[0]].set(W_out.T)
  bo = jnp.zeros((1, D), jnp.float32).at[0, :b_out.shape[0]].set(b_out)
  out_full = _tc2(aggp2, degp, h1, W_l2.T, b_l2[None, :], W_r2.T, wo, bo)
  return out_full[:N, :W_out.shape[0]]


# speedup vs baseline: 1.3800x; 1.3800x over previous
import functools

import jax
import jax.numpy as jnp
from jax import lax
from jax.experimental import pallas as pl
from jax.experimental.pallas import tpu as pltpu
from jax.experimental.pallas import tpu_sc as plsc

NC, NS = 2, 16
NW = NC * NS
N = 10000
NP = 10240
E = 320000
CH = 128
NCHUNK = 80
NSUB = 8
TOTC = NW * NCHUNK
N0 = 144
N1 = 160 - N0
EPT = NCHUNK * CH
EPAD = NW * EPT - E
D = 128
DW = 16
DUMMY_DST = 10008
RPS = NP // NS
BR = 512


def _make_agg(width):
  mesh = plsc.VectorSubcoreMesh(
      core_axis_name="c", subcore_axis_name="s", num_cores=NC,
      num_subcores=NS)

  out_type = jax.ShapeDtypeStruct((NC, NP, width), jnp.float32)
  scratch = [
      pltpu.VMEM((NSUB, CH), jnp.int32),
      pltpu.VMEM((NSUB, CH), jnp.int32),
      pltpu.VMEM((CH, width), jnp.float32),
      pltpu.VMEM((CH, width), jnp.float32),
      pltpu.VMEM_SHARED((NP, width), jnp.float32),
      pltpu.SemaphoreType.DMA,
      pltpu.SemaphoreType.DMA,
      pltpu.SemaphoreType.DMA,
      pltpu.SemaphoreType.DMA,
  ]

  def body(x_hbm, src_hbm, dst_hbm, agg_hbm, src_v, dst_v, rows0, rows1,
           agg_sh, gs0, gs1, ss0, ss1):
    c = lax.axis_index("c")
    s = lax.axis_index("s")
    wid = s * NC + c
    rows = (rows0, rows1)
    gs = (gs0, gs1)
    ss = (ss0, ss1)

    zeros16 = jnp.zeros((16,), jnp.float32)

    def zrow(i, carry):
      for k in range(width // 16):
        rows0[i, pl.ds(k * 16, 16)] = zeros16
      return carry

    lax.fori_loop(0, CH, zrow, 0)

    def zagg(i, carry):
      pltpu.sync_copy(rows0, agg_sh.at[pl.ds(s * RPS + i * CH, CH)])
      return carry

    lax.fori_loop(0, RPS // CH, zagg, 0)
    plsc.subcore_barrier()

    def mk_sup(base):
      def sup(i, carry):
        off = base + i * NSUB
        pltpu.sync_copy(src_hbm.at[pl.ds(off, NSUB)], src_v)
        pltpu.sync_copy(dst_hbm.at[pl.ds(off, NSUB)], dst_v)
        gd = [None] * NSUB
        gd[0] = pltpu.async_copy(x_hbm.at[src_v.at[0]], rows[0], gs[0])
        gd[1] = pltpu.async_copy(x_hbm.at[src_v.at[1]], rows[1], gs[1])
        for k in range(NSUB):
          b = k % 2
          gd[k].wait()
          sd = pltpu.async_copy(rows[b], agg_sh.at[dst_v.at[k]], ss[b],
                                add=True)
          sd.wait()
          if k + 2 < NSUB:
            gd[k + 2] = pltpu.async_copy(x_hbm.at[src_v.at[k + 2]], rows[b],
                                         gs[b])
        return carry
      return sup

    @pl.when(c == 0)
    def _():
      lax.fori_loop(0, N0 // NSUB, mk_sup(s * N0), 0)

    @pl.when(c == 1)
    def _():
      lax.fori_loop(0, N1 // NSUB, mk_sup(NS * N0 + s * N1), 0)

    plsc.subcore_barrier()

    pltpu.sync_copy(agg_sh.at[pl.ds(s * RPS, RPS)],
                    agg_hbm.at[c, pl.ds(s * RPS, RPS)])

  return pl.kernel(
      body, out_type=out_type, mesh=mesh, scratch_types=scratch)


_agg = _make_agg(D)


def _make_deg():
  mesh = plsc.VectorSubcoreMesh(
      core_axis_name="c", subcore_axis_name="s", num_cores=NC,
      num_subcores=NS)

  out_type = jax.ShapeDtypeStruct((NC, NP, D), jnp.float32)
  scratch = [
      pltpu.VMEM((NSUB, CH), jnp.int32),
      pltpu.VMEM((CH, D), jnp.float32),
      pltpu.VMEM_SHARED((NP, D), jnp.float32),
      pltpu.SemaphoreType.DMA,
  ]

  def body(dst_hbm, deg_hbm, dst_v, ones_v, deg_sh, sem):
    c = lax.axis_index("c")
    s = lax.axis_index("s")
    wid = s * NC + c

    zeros16 = jnp.zeros((16,), jnp.float32)
    ones16 = jnp.ones((16,), jnp.float32)

    def zrow(i, carry):
      for k in range(D // 16):
        ones_v[i, pl.ds(k * 16, 16)] = zeros16
      return carry

    lax.fori_loop(0, CH, zrow, 0)

    def zdeg(i, carry):
      pltpu.sync_copy(ones_v, deg_sh.at[pl.ds(s * RPS + i * CH, CH)])
      return carry

    lax.fori_loop(0, RPS // CH, zdeg, 0)

    def frow(i, carry):
      for k in range(D // 16):
        ones_v[i, pl.ds(k * 16, 16)] = ones16
      return carry

    lax.fori_loop(0, CH, frow, 0)
    plsc.subcore_barrier()

    def sup(i, carry):
      pltpu.sync_copy(dst_hbm.at[pl.ds(wid * NCHUNK + i * NSUB, NSUB)], dst_v)
      descs = [
          pltpu.async_copy(ones_v, deg_sh.at[dst_v.at[k]], sem, add=True)
          for k in range(NSUB)
      ]
      for d in descs:
        d.wait()
      return carry

    lax.fori_loop(0, NCHUNK // NSUB, sup, 0)
    plsc.subcore_barrier()

    pltpu.sync_copy(deg_sh.at[pl.ds(s * RPS, RPS)],
                    deg_hbm.at[c, pl.ds(s * RPS, RPS)])

  return pl.kernel(
      body, out_type=out_type, mesh=mesh, scratch_types=scratch)


_deg = _make_deg()


def _layer1_body(aggp_ref, degp_ref, x_ref, wl_ref, bl_ref, wr_ref, out_ref):
  agg = aggp_ref[0] + aggp_ref[1]
  deg = degp_ref[0, :, 0] + degp_ref[1, :, 0]
  inv = 1.0 / jnp.clip(deg, 1.0, None)
  mean = agg * inv[:, None]
  h = (jnp.dot(mean, wl_ref[...], preferred_element_type=jnp.float32)
       + bl_ref[...]
       + jnp.dot(x_ref[...], wr_ref[...], preferred_element_type=jnp.float32))
  out_ref[...] = jnp.maximum(h, 0.0)


def _layer2_body(aggp_ref, degp_ref, x_ref, wl_ref, bl_ref, wr_ref,
                 wo_ref, bo_ref, out_ref):
  agg = aggp_ref[0] + aggp_ref[1]
  deg = degp_ref[0, :, 0] + degp_ref[1, :, 0]
  inv = 1.0 / jnp.clip(deg, 1.0, None)
  mean = agg * inv[:, None]
  h = (jnp.dot(mean, wl_ref[...], preferred_element_type=jnp.float32)
       + bl_ref[...]
       + jnp.dot(x_ref[...], wr_ref[...], preferred_element_type=jnp.float32))
  h = jnp.maximum(h, 0.0)
  out_ref[...] = (jnp.dot(h, wo_ref[...], preferred_element_type=jnp.float32)
                  + bo_ref[...])


def _tc_layer(body, n_extra):
  grid = (NP // BR,)
  in_specs = [
      pl.BlockSpec((NC, BR, D), lambda i: (0, i, 0)),
      pl.BlockSpec((NC, BR, D), lambda i: (0, i, 0)),
      pl.BlockSpec((BR, D), lambda i: (i, 0)),
      pl.BlockSpec((D, D), lambda i: (0, 0)),
      pl.BlockSpec((1, D), lambda i: (0, 0)),
      pl.BlockSpec((D, D), lambda i: (0, 0)),
  ] + [
      pl.BlockSpec((D, D), lambda i: (0, 0)),
      pl.BlockSpec((1, D), lambda i: (0, 0)),
  ][:n_extra]
  return pl.pallas_call(
      body,
      grid=grid,
      in_specs=in_specs,
      out_specs=pl.BlockSpec((BR, D), lambda i: (i, 0)),
      out_shape=jax.ShapeDtypeStruct((NP, D), jnp.float32),
  )


_tc1 = _tc_layer(_layer1_body, 0)
_tc2 = _tc_layer(_layer2_body, 2)


def kernel(x_content, edge_index, edge_type, W_l1, b_l1, W_r1,
           W_l2, b_l2, W_r2, W_out, b_out):
  ei = edge_index.astype(jnp.int32)
  src = jnp.concatenate(
      [ei[0], jnp.zeros((EPAD,), jnp.int32)]).reshape(TOTC, CH)
  dst = jnp.concatenate(
      [ei[1], jnp.full((EPAD,), DUMMY_DST, jnp.int32)]).reshape(TOTC, CH)
  xp = jnp.pad(x_content, ((0, NP - N), (0, 0)))

  degp = _deg(dst)
  aggp1 = _agg(xp, src, dst)
  h1 = _tc1(aggp1, degp, xp, W_l1.T, b_l1[None, :], W_r1.T)
  aggp2 = _agg(h1, src, dst)

  wo = jnp.zeros((D, D), jnp.float32).at[:, :W_out.shape[0]].set(W_out.T)
  bo = jnp.zeros((1, D), jnp.float32).at[0, :b_out.shape[0]].set(b_out)
  out_full = _tc2(aggp2, degp, h1, W_l2.T, b_l2[None, :], W_r2.T, wo, bo)
  return out_full[:N, :W_out.shape[0]]


# speedup vs baseline: 1.3905x; 1.0076x over previous
import functools

import jax
import jax.numpy as jnp
from jax import lax
from jax.experimental import pallas as pl
from jax.experimental.pallas import tpu as pltpu
from jax.experimental.pallas import tpu_sc as plsc

NC, NS = 2, 16
NW = NC * NS
N = 10000
NP = 10240
E = 320000
CH = 128
NCHUNK = 80
NSUB = 8
TOTC = NW * NCHUNK
N0 = 152
N1 = 160 - N0
EPT = NCHUNK * CH
EPAD = NW * EPT - E
D = 128
DW = 16
DUMMY_DST = 10008
RPS = NP // NS
BR = 512


def _make_agg(width):
  mesh = plsc.VectorSubcoreMesh(
      core_axis_name="c", subcore_axis_name="s", num_cores=NC,
      num_subcores=NS)

  out_type = jax.ShapeDtypeStruct((NC, NP, width), jnp.float32)
  scratch = [
      pltpu.VMEM((NSUB, CH), jnp.int32),
      pltpu.VMEM((NSUB, CH), jnp.int32),
      pltpu.VMEM((CH, width), jnp.float32),
      pltpu.VMEM((CH, width), jnp.float32),
      pltpu.VMEM_SHARED((NP, width), jnp.float32),
      pltpu.SemaphoreType.DMA,
      pltpu.SemaphoreType.DMA,
      pltpu.SemaphoreType.DMA,
      pltpu.SemaphoreType.DMA,
  ]

  def body(x_hbm, src_hbm, dst_hbm, agg_hbm, src_v, dst_v, rows0, rows1,
           agg_sh, gs0, gs1, ss0, ss1):
    c = lax.axis_index("c")
    s = lax.axis_index("s")
    wid = s * NC + c
    rows = (rows0, rows1)
    gs = (gs0, gs1)
    ss = (ss0, ss1)

    zeros16 = jnp.zeros((16,), jnp.float32)

    def zrow(i, carry):
      for k in range(width // 16):
        rows0[i, pl.ds(k * 16, 16)] = zeros16
      return carry

    lax.fori_loop(0, CH, zrow, 0)

    def zagg(i, carry):
      pltpu.sync_copy(rows0, agg_sh.at[pl.ds(s * RPS + i * CH, CH)])
      return carry

    lax.fori_loop(0, RPS // CH, zagg, 0)
    plsc.subcore_barrier()

    def mk_sup(base):
      def sup(i, carry):
        off = base + i * NSUB
        pltpu.sync_copy(src_hbm.at[pl.ds(off, NSUB)], src_v)
        pltpu.sync_copy(dst_hbm.at[pl.ds(off, NSUB)], dst_v)
        gd = [None] * NSUB
        gd[0] = pltpu.async_copy(x_hbm.at[src_v.at[0]], rows[0], gs[0])
        gd[1] = pltpu.async_copy(x_hbm.at[src_v.at[1]], rows[1], gs[1])
        for k in range(NSUB):
          b = k % 2
          gd[k].wait()
          sd = pltpu.async_copy(rows[b], agg_sh.at[dst_v.at[k]], ss[b],
                                add=True)
          sd.wait()
          if k + 2 < NSUB:
            gd[k + 2] = pltpu.async_copy(x_hbm.at[src_v.at[k + 2]], rows[b],
                                         gs[b])
        return carry
      return sup

    @pl.when(c == 0)
    def _():
      lax.fori_loop(0, N0 // NSUB, mk_sup(s * N0), 0)

    @pl.when(c == 1)
    def _():
      lax.fori_loop(0, N1 // NSUB, mk_sup(NS * N0 + s * N1), 0)

    plsc.subcore_barrier()

    pltpu.sync_copy(agg_sh.at[pl.ds(s * RPS, RPS)],
                    agg_hbm.at[c, pl.ds(s * RPS, RPS)])

  return pl.kernel(
      body, out_type=out_type, mesh=mesh, scratch_types=scratch)


_agg = _make_agg(D)


def _make_deg():
  mesh = plsc.VectorSubcoreMesh(
      core_axis_name="c", subcore_axis_name="s", num_cores=NC,
      num_subcores=NS)

  out_type = jax.ShapeDtypeStruct((NC, NP, D), jnp.float32)
  scratch = [
      pltpu.VMEM((NSUB, CH), jnp.int32),
      pltpu.VMEM((CH, D), jnp.float32),
      pltpu.VMEM_SHARED((NP, D), jnp.float32),
      pltpu.SemaphoreType.DMA,
  ]

  def body(dst_hbm, deg_hbm, dst_v, ones_v, deg_sh, sem):
    c = lax.axis_index("c")
    s = lax.axis_index("s")
    wid = s * NC + c

    zeros16 = jnp.zeros((16,), jnp.float32)
    ones16 = jnp.ones((16,), jnp.float32)

    def zrow(i, carry):
      for k in range(D // 16):
        ones_v[i, pl.ds(k * 16, 16)] = zeros16
      return carry

    lax.fori_loop(0, CH, zrow, 0)

    def zdeg(i, carry):
      pltpu.sync_copy(ones_v, deg_sh.at[pl.ds(s * RPS + i * CH, CH)])
      return carry

    lax.fori_loop(0, RPS // CH, zdeg, 0)

    def frow(i, carry):
      for k in range(D // 16):
        ones_v[i, pl.ds(k * 16, 16)] = ones16
      return carry

    lax.fori_loop(0, CH, frow, 0)
    plsc.subcore_barrier()

    def sup(i, carry):
      pltpu.sync_copy(dst_hbm.at[pl.ds(wid * NCHUNK + i * NSUB, NSUB)], dst_v)
      descs = [
          pltpu.async_copy(ones_v, deg_sh.at[dst_v.at[k]], sem, add=True)
          for k in range(NSUB)
      ]
      for d in descs:
        d.wait()
      return carry

    lax.fori_loop(0, NCHUNK // NSUB, sup, 0)
    plsc.subcore_barrier()

    pltpu.sync_copy(deg_sh.at[pl.ds(s * RPS, RPS)],
                    deg_hbm.at[c, pl.ds(s * RPS, RPS)])

  return pl.kernel(
      body, out_type=out_type, mesh=mesh, scratch_types=scratch)


_deg = _make_deg()


def _layer1_body(aggp_ref, degp_ref, x_ref, wl_ref, bl_ref, wr_ref, out_ref):
  agg = aggp_ref[0] + aggp_ref[1]
  deg = degp_ref[0, :, 0] + degp_ref[1, :, 0]
  inv = 1.0 / jnp.clip(deg, 1.0, None)
  mean = agg * inv[:, None]
  h = (jnp.dot(mean, wl_ref[...], preferred_element_type=jnp.float32)
       + bl_ref[...]
       + jnp.dot(x_ref[...], wr_ref[...], preferred_element_type=jnp.float32))
  out_ref[...] = jnp.maximum(h, 0.0)


def _layer2_body(aggp_ref, degp_ref, x_ref, wl_ref, bl_ref, wr_ref,
                 wo_ref, bo_ref, out_ref):
  agg = aggp_ref[0] + aggp_ref[1]
  deg = degp_ref[0, :, 0] + degp_ref[1, :, 0]
  inv = 1.0 / jnp.clip(deg, 1.0, None)
  mean = agg * inv[:, None]
  h = (jnp.dot(mean, wl_ref[...], preferred_element_type=jnp.float32)
       + bl_ref[...]
       + jnp.dot(x_ref[...], wr_ref[...], preferred_element_type=jnp.float32))
  h = jnp.maximum(h, 0.0)
  out_ref[...] = (jnp.dot(h, wo_ref[...], preferred_element_type=jnp.float32)
                  + bo_ref[...])


def _tc_layer(body, n_extra):
  grid = (NP // BR,)
  in_specs = [
      pl.BlockSpec((NC, BR, D), lambda i: (0, i, 0)),
      pl.BlockSpec((NC, BR, D), lambda i: (0, i, 0)),
      pl.BlockSpec((BR, D), lambda i: (i, 0)),
      pl.BlockSpec((D, D), lambda i: (0, 0)),
      pl.BlockSpec((1, D), lambda i: (0, 0)),
      pl.BlockSpec((D, D), lambda i: (0, 0)),
  ] + [
      pl.BlockSpec((D, D), lambda i: (0, 0)),
      pl.BlockSpec((1, D), lambda i: (0, 0)),
  ][:n_extra]
  return pl.pallas_call(
      body,
      grid=grid,
      in_specs=in_specs,
      out_specs=pl.BlockSpec((BR, D), lambda i: (i, 0)),
      out_shape=jax.ShapeDtypeStruct((NP, D), jnp.float32),
  )


_tc1 = _tc_layer(_layer1_body, 0)
_tc2 = _tc_layer(_layer2_body, 2)


def kernel(x_content, edge_index, edge_type, W_l1, b_l1, W_r1,
           W_l2, b_l2, W_r2, W_out, b_out):
  ei = edge_index.astype(jnp.int32)
  src = jnp.concatenate(
      [ei[0], jnp.zeros((EPAD,), jnp.int32)]).reshape(TOTC, CH)
  dst = jnp.concatenate(
      [ei[1], jnp.full((EPAD,), DUMMY_DST, jnp.int32)]).reshape(TOTC, CH)
  xp = jnp.pad(x_content, ((0, NP - N), (0, 0)))

  degp = _deg(dst)
  aggp1 = _agg(xp, src, dst)
  h1 = _tc1(aggp1, degp, xp, W_l1.T, b_l1[None, :], W_r1.T)
  aggp2 = _agg(h1, src, dst)

  wo = jnp.zeros((D, D), jnp.float32).at[:, :W_out.shape[0]].set(W_out.T)
  bo = jnp.zeros((1, D), jnp.float32).at[0, :b_out.shape[0]].set(b_out)
  out_full = _tc2(aggp2, degp, h1, W_l2.T, b_l2[None, :], W_r2.T, wo, bo)
  return out_full[:N, :W_out.shape[0]]


# speedup vs baseline: 1.3919x; 1.0010x over previous
import functools

import jax
import jax.numpy as jnp
from jax import lax
from jax.experimental import pallas as pl
from jax.experimental.pallas import tpu as pltpu
from jax.experimental.pallas import tpu_sc as plsc

NC, NS = 2, 16
NW = NC * NS
N = 10000
NP = 10240
E = 320000
CH = 128
NCHUNK = 80
NSUB = 8
TOTC = NW * NCHUNK
N0 = 152
N1 = 160 - N0
EPT = NCHUNK * CH
EPAD = NW * EPT - E
D = 128
DW = 16
DUMMY_DST = 10008
RPS = NP // NS
BR = 512


def _make_agg(width):
  mesh = plsc.VectorSubcoreMesh(
      core_axis_name="c", subcore_axis_name="s", num_cores=NC,
      num_subcores=NS)

  out_type = jax.ShapeDtypeStruct((NC, NP, width), jnp.float32)
  scratch = [
      pltpu.VMEM((2, NSUB, CH), jnp.int32),
      pltpu.VMEM((2, NSUB, CH), jnp.int32),
      pltpu.VMEM((CH, width), jnp.float32),
      pltpu.VMEM((CH, width), jnp.float32),
      pltpu.VMEM_SHARED((NP, width), jnp.float32),
      pltpu.SemaphoreType.DMA,
      pltpu.SemaphoreType.DMA,
      pltpu.SemaphoreType.DMA,
      pltpu.SemaphoreType.DMA,
      pltpu.SemaphoreType.DMA,
      pltpu.SemaphoreType.DMA,
      pltpu.SemaphoreType.DMA,
      pltpu.SemaphoreType.DMA,
  ]

  def body(x_hbm, src_hbm, dst_hbm, agg_hbm, src_v, dst_v, rows0, rows1,
           agg_sh, gs0, gs1, ss0, ss1, sls0, sls1, sld0, sld1):
    sls = (sls0, sls1)
    sld = (sld0, sld1)
    c = lax.axis_index("c")
    s = lax.axis_index("s")
    rows = (rows0, rows1)
    gs = (gs0, gs1)
    ss = (ss0, ss1)

    zeros16 = jnp.zeros((16,), jnp.float32)

    def zrow(i, carry):
      for k in range(width // 16):
        rows0[i, pl.ds(k * 16, 16)] = zeros16
      return carry

    lax.fori_loop(0, CH, zrow, 0)

    zdescs = [
        pltpu.async_copy(rows0, agg_sh.at[pl.ds(s * RPS + i * CH, CH)], gs0)
        for i in range(RPS // CH)
    ]
    for zd in zdescs:
      zd.wait()
    plsc.subcore_barrier()

    def wait_slab(p):
      pltpu.make_async_copy(
          src_hbm.at[pl.ds(0, NSUB)], src_v.at[p], sls[p]).wait()
      pltpu.make_async_copy(
          dst_hbm.at[pl.ds(0, NSUB)], dst_v.at[p], sld[p]).wait()

    def load_slab(p, off):
      pltpu.async_copy(src_hbm.at[pl.ds(off, NSUB)], src_v.at[p], sls[p])
      pltpu.async_copy(dst_hbm.at[pl.ds(off, NSUB)], dst_v.at[p], sld[p])

    def process_slab(p):
      sv = src_v.at[p]
      dv = dst_v.at[p]
      gd = [None] * NSUB
      gd[0] = pltpu.async_copy(x_hbm.at[sv.at[0]], rows[0], gs[0])
      gd[1] = pltpu.async_copy(x_hbm.at[sv.at[1]], rows[1], gs[1])
      for k in range(NSUB):
        b = k % 2
        gd[k].wait()
        sd = pltpu.async_copy(rows[b], agg_sh.at[dv.at[k]], ss[b], add=True)
        sd.wait()
        if k + 2 < NSUB:
          gd[k + 2] = pltpu.async_copy(x_hbm.at[sv.at[k + 2]], rows[b],
                                       gs[b])

    def run(base, nchunks):
      nslab = nchunks // NSUB
      load_slab(0, base)
      load_slab(1, base + min(1, nslab - 1) * NSUB)

      def pair(i, carry):
        for p in (0, 1):
          wait_slab(p)
          process_slab(p)
          off = base + jnp.minimum(2 * i + p + 2, nslab - 1) * NSUB
          load_slab(p, off)
        return carry

      lax.fori_loop(0, nslab // 2, pair, 0)
      if nslab % 2:
        wait_slab(0)
        process_slab(0)
        wait_slab(1)
      else:
        wait_slab(0)
        wait_slab(1)

    @pl.when(c == 0)
    def _():
      run(s * N0, N0)

    @pl.when(c == 1)
    def _():
      run(NS * N0 + s * N1, N1)

    plsc.subcore_barrier()

    pltpu.sync_copy(agg_sh.at[pl.ds(s * RPS, RPS)],
                    agg_hbm.at[c, pl.ds(s * RPS, RPS)])

  return pl.kernel(
      body, out_type=out_type, mesh=mesh, scratch_types=scratch)


_agg = _make_agg(D)


def _make_deg():
  mesh = plsc.VectorSubcoreMesh(
      core_axis_name="c", subcore_axis_name="s", num_cores=NC,
      num_subcores=NS)

  out_type = jax.ShapeDtypeStruct((NC, NP, D), jnp.float32)
  scratch = [
      pltpu.VMEM((NSUB, CH), jnp.int32),
      pltpu.VMEM((CH, D), jnp.float32),
      pltpu.VMEM_SHARED((NP, D), jnp.float32),
      pltpu.SemaphoreType.DMA,
  ]

  def body(dst_hbm, deg_hbm, dst_v, ones_v, deg_sh, sem):
    c = lax.axis_index("c")
    s = lax.axis_index("s")
    wid = s * NC + c

    zeros16 = jnp.zeros((16,), jnp.float32)
    ones16 = jnp.ones((16,), jnp.float32)

    def zrow(i, carry):
      for k in range(D // 16):
        ones_v[i, pl.ds(k * 16, 16)] = zeros16
      return carry

    lax.fori_loop(0, CH, zrow, 0)

    def zdeg(i, carry):
      pltpu.sync_copy(ones_v, deg_sh.at[pl.ds(s * RPS + i * CH, CH)])
      return carry

    lax.fori_loop(0, RPS // CH, zdeg, 0)

    def frow(i, carry):
      for k in range(D // 16):
        ones_v[i, pl.ds(k * 16, 16)] = ones16
      return carry

    lax.fori_loop(0, CH, frow, 0)
    plsc.subcore_barrier()

    def sup(i, carry):
      pltpu.sync_copy(dst_hbm.at[pl.ds(wid * NCHUNK + i * NSUB, NSUB)], dst_v)
      descs = [
          pltpu.async_copy(ones_v, deg_sh.at[dst_v.at[k]], sem, add=True)
          for k in range(NSUB)
      ]
      for d in descs:
        d.wait()
      return carry

    lax.fori_loop(0, NCHUNK // NSUB, sup, 0)
    plsc.subcore_barrier()

    pltpu.sync_copy(deg_sh.at[pl.ds(s * RPS, RPS)],
                    deg_hbm.at[c, pl.ds(s * RPS, RPS)])

  return pl.kernel(
      body, out_type=out_type, mesh=mesh, scratch_types=scratch)


_deg = _make_deg()


def _layer1_body(aggp_ref, degp_ref, x_ref, wl_ref, bl_ref, wr_ref, out_ref):
  agg = aggp_ref[0] + aggp_ref[1]
  deg = degp_ref[0, :, 0] + degp_ref[1, :, 0]
  inv = 1.0 / jnp.clip(deg, 1.0, None)
  mean = agg * inv[:, None]
  h = (jnp.dot(mean, wl_ref[...], preferred_element_type=jnp.float32)
       + bl_ref[...]
       + jnp.dot(x_ref[...], wr_ref[...], preferred_element_type=jnp.float32))
  out_ref[...] = jnp.maximum(h, 0.0)


def _layer2_body(aggp_ref, degp_ref, x_ref, wl_ref, bl_ref, wr_ref,
                 wo_ref, bo_ref, out_ref):
  agg = aggp_ref[0] + aggp_ref[1]
  deg = degp_ref[0, :, 0] + degp_ref[1, :, 0]
  inv = 1.0 / jnp.clip(deg, 1.0, None)
  mean = agg * inv[:, None]
  h = (jnp.dot(mean, wl_ref[...], preferred_element_type=jnp.float32)
       + bl_ref[...]
       + jnp.dot(x_ref[...], wr_ref[...], preferred_element_type=jnp.float32))
  h = jnp.maximum(h, 0.0)
  out_ref[...] = (jnp.dot(h, wo_ref[...], preferred_element_type=jnp.float32)
                  + bo_ref[...])


def _tc_layer(body, n_extra):
  grid = (NP // BR,)
  in_specs = [
      pl.BlockSpec((NC, BR, D), lambda i: (0, i, 0)),
      pl.BlockSpec((NC, BR, D), lambda i: (0, i, 0)),
      pl.BlockSpec((BR, D), lambda i: (i, 0)),
      pl.BlockSpec((D, D), lambda i: (0, 0)),
      pl.BlockSpec((1, D), lambda i: (0, 0)),
      pl.BlockSpec((D, D), lambda i: (0, 0)),
  ] + [
      pl.BlockSpec((D, D), lambda i: (0, 0)),
      pl.BlockSpec((1, D), lambda i: (0, 0)),
  ][:n_extra]
  return pl.pallas_call(
      body,
      grid=grid,
      in_specs=in_specs,
      out_specs=pl.BlockSpec((BR, D), lambda i: (i, 0)),
      out_shape=jax.ShapeDtypeStruct((NP, D), jnp.float32),
  )


_tc1 = _tc_layer(_layer1_body, 0)
_tc2 = _tc_layer(_layer2_body, 2)


def kernel(x_content, edge_index, edge_type, W_l1, b_l1, W_r1,
           W_l2, b_l2, W_r2, W_out, b_out):
  ei = edge_index.astype(jnp.int32)
  src = jnp.concatenate(
      [ei[0], jnp.zeros((EPAD,), jnp.int32)]).reshape(TOTC, CH)
  dst = jnp.concatenate(
      [ei[1], jnp.full((EPAD,), DUMMY_DST, jnp.int32)]).reshape(TOTC, CH)
  xp = jnp.pad(x_content, ((0, NP - N), (0, 0)))

  degp = _deg(dst)
  aggp1 = _agg(xp, src, dst)
  h1 = _tc1(aggp1, degp, xp, W_l1.T, b_l1[None, :], W_r1.T)
  aggp2 = _agg(h1, src, dst)

  wo = jnp.zeros((D, D), jnp.float32).at[:, :W_out.shape[0]].set(W_out.T)
  bo = jnp.zeros((1, D), jnp.float32).at[0, :b_out.shape[0]].set(b_out)
  out_full = _tc2(aggp2, degp, h1, W_l2.T, b_l2[None, :], W_r2.T, wo, bo)
  return out_full[:N, :W_out.shape[0]]


# speedup vs baseline: 1.3928x; 1.0007x over previous
import jax
import jax.numpy as jnp
from jax import lax
from jax.experimental import pallas as pl
from jax.experimental.pallas import tpu as pltpu
from jax.experimental.pallas import tpu_sc as plsc

NC, NS = 2, 16
NW = NC * NS
N = 10000
NP = 10240
E = 320000
CH = 128
NCHUNK = 80
NSUB = 8
TOTC = NW * NCHUNK
N0 = 152
N1 = 160 - N0
EPT = NCHUNK * CH
EPAD = NW * EPT - E
D = 128
DUMMY_DST = 10008
RPS = NP // NS
BR = 512


def _make_agg(width):
  mesh = plsc.VectorSubcoreMesh(
      core_axis_name="c", subcore_axis_name="s", num_cores=NC,
      num_subcores=NS)

  out_type = jax.ShapeDtypeStruct((NC, NP, width), jnp.float32)
  scratch = [
      pltpu.VMEM((2, NSUB, CH), jnp.int32),
      pltpu.VMEM((2, NSUB, CH), jnp.int32),
      pltpu.VMEM((CH, width), jnp.float32),
      pltpu.VMEM((CH, width), jnp.float32),
      pltpu.VMEM_SHARED((NP, width), jnp.float32),
      pltpu.SemaphoreType.DMA,
      pltpu.SemaphoreType.DMA,
      pltpu.SemaphoreType.DMA,
      pltpu.SemaphoreType.DMA,
      pltpu.SemaphoreType.DMA,
      pltpu.SemaphoreType.DMA,
      pltpu.SemaphoreType.DMA,
      pltpu.SemaphoreType.DMA,
  ]

  def body(x_hbm, src_hbm, dst_hbm, agg_hbm, src_v, dst_v, rows0, rows1,
           agg_sh, gs0, gs1, ss0, ss1, sls0, sls1, sld0, sld1):
    sls = (sls0, sls1)
    sld = (sld0, sld1)
    c = lax.axis_index("c")
    s = lax.axis_index("s")
    rows = (rows0, rows1)
    gs = (gs0, gs1)
    ss = (ss0, ss1)

    zeros16 = jnp.zeros((16,), jnp.float32)

    def zrow(i, carry):
      for k in range(width // 16):
        rows0[i, pl.ds(k * 16, 16)] = zeros16
      return carry

    lax.fori_loop(0, CH, zrow, 0)

    zdescs = [
        pltpu.async_copy(rows0, agg_sh.at[pl.ds(s * RPS + i * CH, CH)], gs0)
        for i in range(RPS // CH)
    ]
    for zd in zdescs:
      zd.wait()
    plsc.subcore_barrier()

    def wait_slab(p):
      pltpu.make_async_copy(
          src_hbm.at[pl.ds(0, NSUB)], src_v.at[p], sls[p]).wait()
      pltpu.make_async_copy(
          dst_hbm.at[pl.ds(0, NSUB)], dst_v.at[p], sld[p]).wait()

    def load_slab(p, off):
      pltpu.async_copy(src_hbm.at[pl.ds(off, NSUB)], src_v.at[p], sls[p])
      pltpu.async_copy(dst_hbm.at[pl.ds(off, NSUB)], dst_v.at[p], sld[p])

    def process_slab(p):
      sv = src_v.at[p]
      dv = dst_v.at[p]
      gd = [None] * NSUB
      gd[0] = pltpu.async_copy(x_hbm.at[sv.at[0]], rows[0], gs[0])
      gd[1] = pltpu.async_copy(x_hbm.at[sv.at[1]], rows[1], gs[1])
      for k in range(NSUB):
        b = k % 2
        gd[k].wait()
        sd = pltpu.async_copy(rows[b], agg_sh.at[dv.at[k]], ss[b], add=True)
        sd.wait()
        if k + 2 < NSUB:
          gd[k + 2] = pltpu.async_copy(x_hbm.at[sv.at[k + 2]], rows[b],
                                       gs[b])

    def run(base, nchunks):
      nslab = nchunks // NSUB
      load_slab(0, base)
      load_slab(1, base + min(1, nslab - 1) * NSUB)

      def pair(i, carry):
        for p in (0, 1):
          wait_slab(p)
          process_slab(p)
          off = base + jnp.minimum(2 * i + p + 2, nslab - 1) * NSUB
          load_slab(p, off)
        return carry

      lax.fori_loop(0, nslab // 2, pair, 0)
      if nslab % 2:
        wait_slab(0)
        process_slab(0)
        wait_slab(1)
      else:
        wait_slab(0)
        wait_slab(1)

    @pl.when(c == 0)
    def _():
      run(s * N0, N0)

    @pl.when(c == 1)
    def _():
      run(NS * N0 + s * N1, N1)

    plsc.subcore_barrier()

    pltpu.sync_copy(agg_sh.at[pl.ds(s * RPS, RPS)],
                    agg_hbm.at[c, pl.ds(s * RPS, RPS)])

  return pl.kernel(
      body, out_type=out_type, mesh=mesh, scratch_types=scratch)


_agg = _make_agg(D)


def _make_deg():
  mesh = plsc.VectorSubcoreMesh(
      core_axis_name="c", subcore_axis_name="s", num_cores=NC,
      num_subcores=NS)

  out_type = jax.ShapeDtypeStruct((NC, NP, D), jnp.float32)
  scratch = [
      pltpu.VMEM((NSUB, CH), jnp.int32),
      pltpu.VMEM((CH, D), jnp.float32),
      pltpu.VMEM_SHARED((NP, D), jnp.float32),
      pltpu.SemaphoreType.DMA,
  ]

  def body(dst_hbm, deg_hbm, dst_v, ones_v, deg_sh, sem):
    c = lax.axis_index("c")
    s = lax.axis_index("s")
    wid = s * NC + c

    zeros16 = jnp.zeros((16,), jnp.float32)
    ones16 = jnp.ones((16,), jnp.float32)

    def zrow(i, carry):
      for k in range(D // 16):
        ones_v[i, pl.ds(k * 16, 16)] = zeros16
      return carry

    lax.fori_loop(0, CH, zrow, 0)

    def zdeg(i, carry):
      pltpu.sync_copy(ones_v, deg_sh.at[pl.ds(s * RPS + i * CH, CH)])
      return carry

    lax.fori_loop(0, RPS // CH, zdeg, 0)

    def frow(i, carry):
      for k in range(D // 16):
        ones_v[i, pl.ds(k * 16, 16)] = ones16
      return carry

    lax.fori_loop(0, CH, frow, 0)
    plsc.subcore_barrier()

    def sup(i, carry):
      pltpu.sync_copy(dst_hbm.at[pl.ds(wid * NCHUNK + i * NSUB, NSUB)], dst_v)
      descs = [
          pltpu.async_copy(ones_v, deg_sh.at[dst_v.at[k]], sem, add=True)
          for k in range(NSUB)
      ]
      for d in descs:
        d.wait()
      return carry

    lax.fori_loop(0, NCHUNK // NSUB, sup, 0)
    plsc.subcore_barrier()

    pltpu.sync_copy(deg_sh.at[pl.ds(s * RPS, RPS)],
                    deg_hbm.at[c, pl.ds(s * RPS, RPS)])

  return pl.kernel(
      body, out_type=out_type, mesh=mesh, scratch_types=scratch)


_deg = _make_deg()


def _layer1_body(aggp_ref, degp_ref, x_ref, wl_ref, bl_ref, wr_ref, out_ref):
  agg = aggp_ref[0] + aggp_ref[1]
  deg = degp_ref[0, :, 0] + degp_ref[1, :, 0]
  inv = 1.0 / jnp.clip(deg, 1.0, None)
  mean = agg * inv[:, None]
  h = (jnp.dot(mean, wl_ref[...], preferred_element_type=jnp.float32)
       + bl_ref[...]
       + jnp.dot(x_ref[...], wr_ref[...], preferred_element_type=jnp.float32))
  out_ref[...] = jnp.maximum(h, 0.0)


def _layer2_body(aggp_ref, degp_ref, x_ref, wl_ref, bl_ref, wr_ref,
                 wo_ref, bo_ref, out_ref):
  agg = aggp_ref[0] + aggp_ref[1]
  deg = degp_ref[0, :, 0] + degp_ref[1, :, 0]
  inv = 1.0 / jnp.clip(deg, 1.0, None)
  mean = agg * inv[:, None]
  h = (jnp.dot(mean, wl_ref[...], preferred_element_type=jnp.float32)
       + bl_ref[...]
       + jnp.dot(x_ref[...], wr_ref[...], preferred_element_type=jnp.float32))
  h = jnp.maximum(h, 0.0)
  out_ref[...] = (jnp.dot(h, wo_ref[...], preferred_element_type=jnp.float32)
                  + bo_ref[...])


def _tc_layer(body, n_extra):
  grid = (NP // BR,)
  in_specs = [
      pl.BlockSpec((NC, BR, D), lambda i: (0, i, 0)),
      pl.BlockSpec((NC, BR, D), lambda i: (0, i, 0)),
      pl.BlockSpec((BR, D), lambda i: (i, 0)),
      pl.BlockSpec((D, D), lambda i: (0, 0)),
      pl.BlockSpec((1, D), lambda i: (0, 0)),
      pl.BlockSpec((D, D), lambda i: (0, 0)),
  ] + [
      pl.BlockSpec((D, D), lambda i: (0, 0)),
      pl.BlockSpec((1, D), lambda i: (0, 0)),
  ][:n_extra]
  return pl.pallas_call(
      body,
      grid=grid,
      in_specs=in_specs,
      out_specs=pl.BlockSpec((BR, D), lambda i: (i, 0)),
      out_shape=jax.ShapeDtypeStruct((NP, D), jnp.float32),
  )


_tc1 = _tc_layer(_layer1_body, 0)
_tc2 = _tc_layer(_layer2_body, 2)


def kernel(x_content, edge_index, edge_type, W_l1, b_l1, W_r1,
           W_l2, b_l2, W_r2, W_out, b_out):
  ei = edge_index.astype(jnp.int32)
  src = jnp.concatenate(
      [ei[0], jnp.zeros((EPAD,), jnp.int32)]).reshape(TOTC, CH)
  dst = jnp.concatenate(
      [ei[1], jnp.full((EPAD,), DUMMY_DST, jnp.int32)]).reshape(TOTC, CH)
  xp = jnp.pad(x_content, ((0, NP - N), (0, 0)))

  degp = _deg(dst)
  aggp1 = _agg(xp, src, dst)
  h1 = _tc1(aggp1, degp, xp, W_l1.T, b_l1[None, :], W_r1.T)
  aggp2 = _agg(h1, src, dst)

  wo = jnp.zeros((D, D), jnp.float32).at[:, :W_out.shape[0]].set(W_out.T)
  bo = jnp.zeros((1, D), jnp.float32).at[0, :b_out.shape[0]].set(b_out)
  out_full = _tc2(aggp2, degp, h1, W_l2.T, b_l2[None, :], W_r2.T, wo, bo)
  return out_full[:N, :W_out.shape[0]]
